# baseline (speedup 1.0000x reference)
import functools

import jax
import jax.numpy as jnp
from jax import lax
from jax.experimental import pallas as pl
from jax.experimental.pallas import tpu as pltpu
from jax.experimental.pallas import tpu_sc as plsc

B = 4096
H = 200
D = 32
H2 = 2 * H
GSZ = (104, 104, 104, 88)
NBUF = 4
RPI = 20
NACC = 4

_info = plsc.get_sparse_core_info()
NC, NS, L = _info.num_cores, _info.num_subcores, _info.num_lanes
NW = NC * NS
BPW = B // NW
IPW = BPW * H
OPW = BPW * D

_mesh = plsc.VectorSubcoreMesh(core_axis_name="c", subcore_axis_name="s")


@functools.partial(
    pl.kernel,
    mesh=_mesh,
    out_type=jax.ShapeDtypeStruct((B * D,), jnp.float32),
    compiler_params=pltpu.CompilerParams(
        use_tc_tiling_on_sc=False, needs_layout_passes=False
    ),
    scratch_types=[
        pltpu.VMEM((IPW,), jnp.int32),
        pltpu.VMEM((BPW * H2,), jnp.int32),
        [pltpu.VMEM((H2, L), jnp.float32) for _ in range(NBUF)],
        pltpu.VMEM((OPW,), jnp.float32),
        [pltpu.SemaphoreType.DMA for _ in range(NBUF)],
    ],
)
def _encode(x_hbm, table_hbm, out_hbm, idx_v, idx2_v, rows, out_v, sems):
    wid = lax.axis_index("s") * NC + lax.axis_index("c")

    pltpu.sync_copy(x_hbm.at[pl.ds(wid * IPW, IPW)], idx_v)

    lane = lax.iota(jnp.int32, L)
    even = 2 * lane
    odd = even + 1

    def expand(g, _):
        v = idx_v[pl.ds(g * L, L)]
        a = v + v
        base = 2 * g * L
        plsc.store_scatter(idx2_v, [base + even], a)
        plsc.store_scatter(idx2_v, [base + odd], a + 1)
        return 0

    lax.fori_loop(0, IPW // L, expand, 0)

    def start_gather(i, b):
        off = 0
        for g in GSZ:
            pltpu.async_copy(
                table_hbm.at[idx2_v.at[pl.ds(i * H2 + off, g)]],
                rows[b].at[pl.ds(off, g)],
                sems[b],
            )
            off += g

    def wait_gather(i, b):
        off = 0
        for g in GSZ:
            pltpu.make_async_copy(
                table_hbm.at[idx2_v.at[pl.ds(i * H2 + off, g)]],
                rows[b].at[pl.ds(off, g)],
                sems[b],
            ).wait()
            off += g

    def reduce_row(i, buf):
        zero = jnp.zeros((L,), jnp.float32)

        def body(j, accs):
            accs = list(accs)
            for r in range(RPI):
                row = RPI * j + r
                lo, hi = accs[r % NACC]
                lo = lo + buf[2 * row, pl.ds(0, L)]
                hi = hi + buf[2 * row + 1, pl.ds(0, L)]
                accs[r % NACC] = (lo, hi)
            return tuple(accs)

        accs = lax.fori_loop(0, H // RPI, body, tuple((zero, zero) for _ in range(NACC)))
        lo = accs[0][0] + accs[1][0] + accs[2][0] + accs[3][0]
        hi = accs[0][1] + accs[1][1] + accs[2][1] + accs[3][1]
        scale = jnp.float32(1.0 / H)
        out_v[pl.ds(i * D, L)] = lo * scale
        out_v[pl.ds(i * D + L, L)] = hi * scale

    for b in range(NBUF):
        start_gather(b, b)

    def outer(k, _):
        i0 = NBUF * k
        for b in range(NBUF):
            wait_gather(i0 + b, b)
            reduce_row(i0 + b, rows[b])
            start_gather(i0 + b + NBUF, b)
        return 0

    lax.fori_loop(0, BPW // NBUF - 1, outer, 0)

    for b in range(NBUF):
        i = BPW - NBUF + b
        wait_gather(i, b)
        reduce_row(i, rows[b])

    pltpu.sync_copy(out_v, out_hbm.at[pl.ds(wid * OPW, OPW)])


def kernel(x, table):
    flat = _encode(x.astype(jnp.int32).reshape(B * H), table.reshape(2 * 1000000, L))
    return flat.reshape(B, D)
